# baseline (speedup 1.0000x reference)
import functools

import jax
import jax.numpy as jnp
import numpy as np
from jax import lax
from jax.experimental import pallas as pl
from jax.experimental.pallas import tpu as pltpu
from jax.experimental.pallas import tpu_sc as plsc

B, N, CD, R = 4, 131072, 32, 256
HW = R * R
TOT = B * N
NW = 32
PTS = TOT // NW
TILES_PER_BATCH = N // PTS

INV_SCALE = np.float32(1.0 / (1.0 + 0.1 + 10e-4))
K1 = np.float32(INV_SCALE * np.float32(R - 1))
K2 = np.float32(0.5 * (R - 1))
GMAX = np.float32(np.float32(1.0 - 10e-6) * np.float32(R - 1))
MASKHI = np.int32(-65536)


YR = 32


def _contract_body(w_ref, x0_ref, x1_ref, x2_ref, o0_ref, o1_ref, o2_ref):
    for x_ref, o_ref in ((x0_ref, o0_ref), (x1_ref, o1_ref), (x2_ref, o2_ref)):
        xs = x_ref[0]
        s = w_ref[0, 0] * xs[0]
        for c in range(1, CD):
            s = s + w_ref[0, c] * xs[c]
        nxt = jnp.concatenate([s[:, 1:], s[:, -1:]], axis=1)
        lo = lax.bitcast_convert_type(
            s.astype(jnp.bfloat16), jnp.uint16).astype(jnp.uint32)
        hi = lax.bitcast_convert_type(
            nxt.astype(jnp.bfloat16), jnp.uint16).astype(jnp.uint32)
        packed = lax.bitcast_convert_type(
            lo | (hi << jnp.uint32(16)), jnp.int32)
        o_ref[...] = packed.reshape(o_ref.shape)


def _contract_planes(w32, xz, xy, yz):
    grid = (B, R // YR)
    x_spec = pl.BlockSpec((1, CD, YR, R), lambda bi, j: (bi, 0, j, 0))
    o_spec = pl.BlockSpec((1, 1, YR * R), lambda bi, j: (bi, 0, j))
    return pl.pallas_call(
        _contract_body,
        grid=grid,
        in_specs=[pl.BlockSpec(memory_space=pltpu.MemorySpace.SMEM),
                  x_spec, x_spec, x_spec],
        out_specs=[o_spec, o_spec, o_spec],
        out_shape=[jax.ShapeDtypeStruct((B, 1, HW), jnp.int32)] * 3,
    )(w32, xz, xy, yz)


def _grid_coord(q):
    g = q * K1 + K2
    g = jnp.minimum(g, GMAX)
    g = jnp.maximum(g, np.float32(0.0))
    xi = g.astype(jnp.int32)
    return xi, g - xi.astype(jnp.float32)


def _sc_sample(sxz, sxy, syz, pt, head_hbm, out,
               plane_buf, ca, cb, acc, head_buf, sem):
    nc = 2
    wid = lax.axis_index("s") * nc + lax.axis_index("c")
    base = wid * PTS
    b = wid // TILES_PER_BATCH
    descs = (
        pltpu.async_copy(head_hbm, head_buf, sem),
        pltpu.async_copy(pt.at[pl.ds(0 * TOT + base, PTS)], ca, sem),
        pltpu.async_copy(pt.at[pl.ds(2 * TOT + base, PTS)], cb, sem),
        pltpu.async_copy(sxz.at[pl.ds(b * HW, HW)], plane_buf, sem),
    )
    for d in descs:
        d.wait()
    cfg = (
        (sxz, None, ca, 0, cb, 2),
        (sxy, (cb, 1), ca, 0, cb, 1),
        (syz, (ca, 2), cb, 1, ca, 2),
    )
    for pi, (sref, newload, xbuf, dx, ybuf, dy) in enumerate(cfg):
        if pi > 0:
            dst, coord = newload
            descs = (
                pltpu.async_copy(
                    pt.at[pl.ds(coord * TOT + base, PTS)], dst, sem),
                pltpu.async_copy(sref.at[pl.ds(b * HW, HW)], plane_buf, sem),
            )
            for d in descs:
                d.wait()
        wx_h = head_buf[dx]
        wy_h = head_buf[dy]
        wbias = head_buf[3]

        @plsc.parallel_loop(0, PTS // 16, unroll=8)
        def _(i, pi=pi, xbuf=xbuf, ybuf=ybuf,
              wx_h=wx_h, wy_h=wy_h, wbias=wbias):
            sl = pl.ds(i * 16, 16)
            px = xbuf[sl]
            py = ybuf[sl]
            xa, fx = _grid_coord(px)
            ya, fy = _grid_coord(py)
            idx = ya * R + xa
            w0 = plsc.load_gather(plane_buf, [idx])
            w1 = plsc.load_gather(plane_buf, [idx + R])
            s00 = plsc.bitcast(lax.shift_left(w0, 16), jnp.float32)
            s01 = plsc.bitcast(lax.bitwise_and(w0, MASKHI), jnp.float32)
            s10 = plsc.bitcast(lax.shift_left(w1, 16), jnp.float32)
            s11 = plsc.bitcast(lax.bitwise_and(w1, MASKHI), jnp.float32)
            top = s00 + fx * (s01 - s00)
            bot = s10 + fx * (s11 - s10)
            val = top + fy * (bot - top)
            val = val + wx_h * px + wy_h * py
            if pi == 0:
                acc[sl] = val + wbias
            else:
                acc[sl] += val

    pltpu.sync_copy(acc, out.at[pl.ds(base, PTS)])


_sc_sample_call = functools.partial(
    pl.kernel,
    out_type=jax.ShapeDtypeStruct((TOT,), jnp.float32),
    mesh=plsc.VectorSubcoreMesh(core_axis_name="c", subcore_axis_name="s"),
    scratch_types=[
        pltpu.VMEM((HW,), jnp.int32),
        pltpu.VMEM((PTS,), jnp.float32),
        pltpu.VMEM((PTS,), jnp.float32),
        pltpu.VMEM((PTS,), jnp.float32),
        pltpu.VMEM((4, 16), jnp.float32),
        pltpu.SemaphoreType.DMA,
    ],
    compiler_params=pltpu.CompilerParams(needs_layout_passes=False),
)(_sc_sample)


@jax.jit
def kernel(p, c_plane_xz, c_plane_xy, c_plane_yz, fc_w, fc_b):
    w32 = fc_w[:, :CD]
    s_xz, s_xy, s_yz = _contract_planes(
        w32, c_plane_xz, c_plane_xy, c_plane_yz)
    pt = jnp.transpose(p.reshape(TOT, 3)).reshape(3 * TOT)
    head_vals = jnp.concatenate([fc_w[0, CD:CD + 3] * 0.5, fc_b])
    head = jnp.broadcast_to(head_vals[:, None], (4, 16))
    out = _sc_sample_call(
        s_xz.reshape(B * HW), s_xy.reshape(B * HW), s_yz.reshape(B * HW),
        pt, head)
    return out.reshape(B, N)


# speedup vs baseline: 1.0580x; 1.0580x over previous
import functools

import jax
import jax.numpy as jnp
import numpy as np
from jax import lax
from jax.experimental import pallas as pl
from jax.experimental.pallas import tpu as pltpu
from jax.experimental.pallas import tpu_sc as plsc

B, N, CD, R = 4, 131072, 32, 256
HW = R * R
TOT = B * N
NW = 32
PTS = TOT // NW
TILES_PER_BATCH = N // PTS

INV_SCALE = np.float32(1.0 / (1.0 + 0.1 + 10e-4))
K1 = np.float32(INV_SCALE * np.float32(R - 1))
K2 = np.float32(0.5 * (R - 1))
GMAX = np.float32(np.float32(1.0 - 10e-6) * np.float32(R - 1))
MASKHI = np.int32(-65536)


YR = 64


def _contract_body(w_ref, x0_ref, x1_ref, x2_ref, o0_ref, o1_ref, o2_ref):
    for x_ref, o_ref in ((x0_ref, o0_ref), (x1_ref, o1_ref), (x2_ref, o2_ref)):
        xs = x_ref[0]
        s = w_ref[0, 0] * xs[0]
        for c in range(1, CD):
            s = s + w_ref[0, c] * xs[c]
        nxt = jnp.concatenate([s[:, 1:], s[:, -1:]], axis=1)
        lo = lax.bitcast_convert_type(
            s.astype(jnp.bfloat16), jnp.uint16).astype(jnp.uint32)
        hi = lax.bitcast_convert_type(
            nxt.astype(jnp.bfloat16), jnp.uint16).astype(jnp.uint32)
        packed = lax.bitcast_convert_type(
            lo | (hi << jnp.uint32(16)), jnp.int32)
        o_ref[...] = packed.reshape(o_ref.shape)


def _contract_planes(w32, xz, xy, yz):
    grid = (B, R // YR)
    x_spec = pl.BlockSpec((1, CD, YR, R), lambda bi, j: (bi, 0, j, 0))
    o_spec = pl.BlockSpec((1, 1, YR * R), lambda bi, j: (bi, 0, j))
    return pl.pallas_call(
        _contract_body,
        grid=grid,
        in_specs=[pl.BlockSpec(memory_space=pltpu.MemorySpace.SMEM),
                  x_spec, x_spec, x_spec],
        out_specs=[o_spec, o_spec, o_spec],
        out_shape=[jax.ShapeDtypeStruct((B, 1, HW), jnp.int32)] * 3,
    )(w32, xz, xy, yz)


def _grid_coord(q):
    g = q * K1 + K2
    g = jnp.minimum(g, GMAX)
    g = jnp.maximum(g, np.float32(0.0))
    xi = g.astype(jnp.int32)
    return xi, g - xi.astype(jnp.float32)


def _sc_sample(sxz, sxy, syz, pt, head_hbm, out,
               plane_buf, ca, cb, acc, head_buf):
    nc = 2
    wid = lax.axis_index("s") * nc + lax.axis_index("c")
    base = wid * PTS
    b = wid // TILES_PER_BATCH
    pltpu.sync_copy(head_hbm, head_buf)
    pltpu.sync_copy(pt.at[pl.ds(0 * TOT + base, PTS)], ca)
    pltpu.sync_copy(pt.at[pl.ds(2 * TOT + base, PTS)], cb)
    cfg = (
        (sxz, None, ca, 0, cb, 2),
        (sxy, (cb, 1), ca, 0, cb, 1),
        (syz, (ca, 2), cb, 1, ca, 2),
    )
    for pi, (sref, newload, xbuf, dx, ybuf, dy) in enumerate(cfg):
        if newload is not None:
            dst, coord = newload
            pltpu.sync_copy(pt.at[pl.ds(coord * TOT + base, PTS)], dst)
        pltpu.sync_copy(sref.at[pl.ds(b * HW, HW)], plane_buf)
        wx_h = head_buf[dx]
        wy_h = head_buf[dy]
        wbias = head_buf[3]

        @plsc.parallel_loop(0, PTS // 16, unroll=4)
        def _(i, pi=pi, xbuf=xbuf, ybuf=ybuf,
              wx_h=wx_h, wy_h=wy_h, wbias=wbias):
            sl = pl.ds(i * 16, 16)
            px = xbuf[sl]
            py = ybuf[sl]
            xa, fx = _grid_coord(px)
            ya, fy = _grid_coord(py)
            idx = ya * R + xa
            w0 = plsc.load_gather(plane_buf, [idx])
            w1 = plsc.load_gather(plane_buf, [idx + R])
            s00 = plsc.bitcast(lax.shift_left(w0, 16), jnp.float32)
            s01 = plsc.bitcast(lax.bitwise_and(w0, MASKHI), jnp.float32)
            s10 = plsc.bitcast(lax.shift_left(w1, 16), jnp.float32)
            s11 = plsc.bitcast(lax.bitwise_and(w1, MASKHI), jnp.float32)
            top = s00 + fx * (s01 - s00)
            bot = s10 + fx * (s11 - s10)
            val = top + fy * (bot - top)
            val = val + wx_h * px + wy_h * py
            if pi == 0:
                acc[sl] = val + wbias
            else:
                acc[sl] += val

    pltpu.sync_copy(acc, out.at[pl.ds(base, PTS)])


_sc_sample_call = functools.partial(
    pl.kernel,
    out_type=jax.ShapeDtypeStruct((TOT,), jnp.float32),
    mesh=plsc.VectorSubcoreMesh(core_axis_name="c", subcore_axis_name="s"),
    scratch_types=[
        pltpu.VMEM((HW,), jnp.int32),
        pltpu.VMEM((PTS,), jnp.float32),
        pltpu.VMEM((PTS,), jnp.float32),
        pltpu.VMEM((PTS,), jnp.float32),
        pltpu.VMEM((4, 16), jnp.float32),
    ],
    compiler_params=pltpu.CompilerParams(needs_layout_passes=False),
)(_sc_sample)


@jax.jit
def kernel(p, c_plane_xz, c_plane_xy, c_plane_yz, fc_w, fc_b):
    w32 = fc_w[:, :CD]
    s_xz, s_xy, s_yz = _contract_planes(
        w32, c_plane_xz, c_plane_xy, c_plane_yz)
    pt = jnp.transpose(p.reshape(TOT, 3)).reshape(3 * TOT)
    head_vals = jnp.concatenate([fc_w[0, CD:CD + 3] * 0.5, fc_b])
    head = jnp.broadcast_to(head_vals[:, None], (4, 16))
    out = _sc_sample_call(
        s_xz.reshape(B * HW), s_xy.reshape(B * HW), s_yz.reshape(B * HW),
        pt, head)
    return out.reshape(B, N)


# speedup vs baseline: 1.0641x; 1.0057x over previous
import functools

import jax
import jax.numpy as jnp
import numpy as np
from jax import lax
from jax.experimental import pallas as pl
from jax.experimental.pallas import tpu as pltpu
from jax.experimental.pallas import tpu_sc as plsc

B, N, CD, R = 4, 131072, 32, 256
HW = R * R
TOT = B * N
NW = 32
PTS = TOT // NW
TILES_PER_BATCH = N // PTS

INV_SCALE = np.float32(1.0 / (1.0 + 0.1 + 10e-4))
K1 = np.float32(INV_SCALE * np.float32(R - 1))
K2 = np.float32(0.5 * (R - 1))
GMAX = np.float32(np.float32(1.0 - 10e-6) * np.float32(R - 1))
MASKHI = np.int32(-65536)


YR = 128


def _contract_body(w_ref, x0_ref, x1_ref, x2_ref, o0_ref, o1_ref, o2_ref):
    for x_ref, o_ref in ((x0_ref, o0_ref), (x1_ref, o1_ref), (x2_ref, o2_ref)):
        xs = x_ref[0]
        s = w_ref[0, 0] * xs[0]
        for c in range(1, CD):
            s = s + w_ref[0, c] * xs[c]
        nxt = jnp.concatenate([s[:, 1:], s[:, -1:]], axis=1)
        lo = lax.bitcast_convert_type(
            s.astype(jnp.bfloat16), jnp.uint16).astype(jnp.uint32)
        hi = lax.bitcast_convert_type(
            nxt.astype(jnp.bfloat16), jnp.uint16).astype(jnp.uint32)
        packed = lax.bitcast_convert_type(
            lo | (hi << jnp.uint32(16)), jnp.int32)
        o_ref[...] = packed.reshape(o_ref.shape)


def _contract_planes(w32, xz, xy, yz):
    grid = (B, R // YR)
    x_spec = pl.BlockSpec((1, CD, YR, R), lambda bi, j: (bi, 0, j, 0))
    o_spec = pl.BlockSpec((1, 1, YR * R), lambda bi, j: (bi, 0, j))
    return pl.pallas_call(
        _contract_body,
        grid=grid,
        in_specs=[pl.BlockSpec(memory_space=pltpu.MemorySpace.SMEM),
                  x_spec, x_spec, x_spec],
        out_specs=[o_spec, o_spec, o_spec],
        out_shape=[jax.ShapeDtypeStruct((B, 1, HW), jnp.int32)] * 3,
    )(w32, xz, xy, yz)


def _grid_coord(q):
    g = q * K1 + K2
    g = jnp.minimum(g, GMAX)
    g = jnp.maximum(g, np.float32(0.0))
    xi = g.astype(jnp.int32)
    return xi, g - xi.astype(jnp.float32)


def _sc_sample(sxz, sxy, syz, pt, head_hbm, out,
               plane_buf, ca, cb, acc, head_buf):
    nc = 2
    wid = lax.axis_index("s") * nc + lax.axis_index("c")
    base = wid * PTS
    b = wid // TILES_PER_BATCH
    pltpu.sync_copy(head_hbm, head_buf)
    pltpu.sync_copy(pt.at[pl.ds(0 * TOT + base, PTS)], ca)
    pltpu.sync_copy(pt.at[pl.ds(2 * TOT + base, PTS)], cb)
    cfg = (
        (sxz, None, ca, 0, cb, 2),
        (sxy, (cb, 1), ca, 0, cb, 1),
        (syz, (ca, 2), cb, 1, ca, 2),
    )
    for pi, (sref, newload, xbuf, dx, ybuf, dy) in enumerate(cfg):
        if newload is not None:
            dst, coord = newload
            pltpu.sync_copy(pt.at[pl.ds(coord * TOT + base, PTS)], dst)
        pltpu.sync_copy(sref.at[pl.ds(b * HW, HW)], plane_buf)
        wx_h = head_buf[dx]
        wy_h = head_buf[dy]
        wbias = head_buf[3]

        @plsc.parallel_loop(0, PTS // 16, unroll=4)
        def _(i, pi=pi, xbuf=xbuf, ybuf=ybuf,
              wx_h=wx_h, wy_h=wy_h, wbias=wbias):
            sl = pl.ds(i * 16, 16)
            px = xbuf[sl]
            py = ybuf[sl]
            xa, fx = _grid_coord(px)
            ya, fy = _grid_coord(py)
            idx = ya * R + xa
            w0 = plsc.load_gather(plane_buf, [idx])
            w1 = plsc.load_gather(plane_buf, [idx + R])
            s00 = plsc.bitcast(lax.shift_left(w0, 16), jnp.float32)
            s01 = plsc.bitcast(lax.bitwise_and(w0, MASKHI), jnp.float32)
            s10 = plsc.bitcast(lax.shift_left(w1, 16), jnp.float32)
            s11 = plsc.bitcast(lax.bitwise_and(w1, MASKHI), jnp.float32)
            top = s00 + fx * (s01 - s00)
            bot = s10 + fx * (s11 - s10)
            val = top + fy * (bot - top)
            val = val + wx_h * px + wy_h * py
            if pi == 0:
                acc[sl] = val + wbias
            else:
                acc[sl] += val

    pltpu.sync_copy(acc, out.at[pl.ds(base, PTS)])


_sc_sample_call = functools.partial(
    pl.kernel,
    out_type=jax.ShapeDtypeStruct((TOT,), jnp.float32),
    mesh=plsc.VectorSubcoreMesh(core_axis_name="c", subcore_axis_name="s"),
    scratch_types=[
        pltpu.VMEM((HW,), jnp.int32),
        pltpu.VMEM((PTS,), jnp.float32),
        pltpu.VMEM((PTS,), jnp.float32),
        pltpu.VMEM((PTS,), jnp.float32),
        pltpu.VMEM((4, 16), jnp.float32),
    ],
    compiler_params=pltpu.CompilerParams(needs_layout_passes=False),
)(_sc_sample)


@jax.jit
def kernel(p, c_plane_xz, c_plane_xy, c_plane_yz, fc_w, fc_b):
    w32 = fc_w[:, :CD]
    s_xz, s_xy, s_yz = _contract_planes(
        w32, c_plane_xz, c_plane_xy, c_plane_yz)
    pt = jnp.transpose(p.reshape(TOT, 3)).reshape(3 * TOT)
    head_vals = jnp.concatenate([fc_w[0, CD:CD + 3] * 0.5, fc_b])
    head = jnp.broadcast_to(head_vals[:, None], (4, 16))
    out = _sc_sample_call(
        s_xz.reshape(B * HW), s_xy.reshape(B * HW), s_yz.reshape(B * HW),
        pt, head)
    return out.reshape(B, N)
